# Initial kernel scaffold; baseline (speedup 1.0000x reference)
import jax
import jax.numpy as jnp
from jax.experimental import pallas as pl


def kernel(x_graph_1, x_graph_2, params, edge_index_g1g1, edge_index_g2g2, edge_index_g1g2, edge_index_g2g1, batch_graph_1, batch_graph_2):
    raise NotImplementedError("write your pallas kernel here")


# speedup vs baseline: 6.9199x; 6.9199x over previous
import functools

import jax
import jax.numpy as jnp
from jax import lax
from jax.experimental import pallas as pl
from jax.experimental.pallas import tpu as pltpu
from jax.experimental.pallas import tpu_sc as plsc

N = 10000
B = 16
H = 256
DIN = 128
NP = 10112
NC, NS, LANES = 2, 16, 16
TN = 1000
EPAD = 4096
NROW = NP
RPT = NROW // NS

_f32 = jnp.float32
_i32 = jnp.int32


def _mm_stats_body(x_ref, w_ref, b_ref, o_ref, st_ref):
    i = pl.program_id(0)
    h = lax.dot_general(x_ref[...], w_ref[...], (((1,), (0,)), ((), ())),
                        preferred_element_type=_f32) + b_ref[...]
    o_ref[...] = h

    @pl.when(i == 0)
    def _():
        st_ref[...] = jnp.zeros_like(st_ref)

    s1 = jnp.sum(h, axis=0, keepdims=True)
    s2 = jnp.sum(h * h, axis=0, keepdims=True)
    st_ref[0:1, :] += s1
    st_ref[1:2, :] += s2


def _mm_stats(x, w, b):
    n, k = x.shape
    c = w.shape[1]
    grid = (n // TN,)
    return pl.pallas_call(
        _mm_stats_body,
        grid=grid,
        in_specs=[
            pl.BlockSpec((TN, k), lambda i: (i, 0)),
            pl.BlockSpec((k, c), lambda i: (0, 0)),
            pl.BlockSpec((1, c), lambda i: (0, 0)),
        ],
        out_specs=[
            pl.BlockSpec((TN, c), lambda i: (i, 0)),
            pl.BlockSpec((8, c), lambda i: (0, 0)),
        ],
        out_shape=[
            jax.ShapeDtypeStruct((n, c), _f32),
            jax.ShapeDtypeStruct((8, c), _f32),
        ],
    )(x, w, b.reshape(1, c))


def _conv_mm_body(x_ref, w_ref, deg_ref, asv_ref, ad_ref,
                  gcn_ref, gat_ref, sd_ref):
    h = lax.dot_general(x_ref[...], w_ref[...], (((1,), (0,)), ((), ())),
                        preferred_element_type=_f32)
    dinv = lax.rsqrt(deg_ref[...])
    hg = h[:, 0:256] * dinv
    gcn_ref[0] = hg[:, 0:128]
    gcn_ref[1] = hg[:, 128:256]
    hs = h[:, 256:512]
    gat_ref[0] = hs[:, 0:128]
    gat_ref[1] = hs[:, 128:256]
    dn = (((1,), (0,)), ((), ()))
    s = lax.dot_general(hs, asv_ref[...], dn, preferred_element_type=_f32,
                        precision=lax.Precision.HIGHEST)
    dcol = lax.dot_general(h[:, 512:768], ad_ref[...], dn,
                           preferred_element_type=_f32,
                           precision=lax.Precision.HIGHEST)
    zero = jnp.zeros((s.shape[0], 126), _f32)
    sd_ref[...] = jnp.concatenate([s, dcol, zero], axis=1)


def _conv_mm(x, wbig, deg, asv, ad):
    n = x.shape[0]
    grid = (n // TN,)
    return pl.pallas_call(
        _conv_mm_body,
        grid=grid,
        in_specs=[
            pl.BlockSpec((TN, H), lambda i: (i, 0)),
            pl.BlockSpec((H, 768), lambda i: (0, 0)),
            pl.BlockSpec((TN, 1), lambda i: (i, 0)),
            pl.BlockSpec((H, 1), lambda i: (0, 0)),
            pl.BlockSpec((H, 1), lambda i: (0, 0)),
        ],
        out_specs=[
            pl.BlockSpec((2, TN, 128), lambda i: (0, i, 0)),
            pl.BlockSpec((2, TN, 128), lambda i: (0, i, 0)),
            pl.BlockSpec((TN, 128), lambda i: (i, 0)),
        ],
        out_shape=[
            jax.ShapeDtypeStruct((2, n, 128), _f32),
            jax.ShapeDtypeStruct((2, n, 128), _f32),
            jax.ShapeDtypeStruct((n, 128), _f32),
        ],
    )(x, wbig, deg, asv.reshape(H, 1), ad.reshape(H, 1))


def _affine_body(relu, h_ref, a_ref, c_ref, o_ref):
    y = h_ref[...] * a_ref[...] + c_ref[...]
    if relu:
        y = jnp.maximum(y, 0.0)
    o_ref[...] = y


def _affine(h, a, c, relu):
    n, d = h.shape
    return pl.pallas_call(
        functools.partial(_affine_body, relu),
        grid=(n // TN,),
        in_specs=[
            pl.BlockSpec((TN, d), lambda i: (i, 0)),
            pl.BlockSpec((1, d), lambda i: (0, 0)),
            pl.BlockSpec((1, d), lambda i: (0, 0)),
        ],
        out_specs=pl.BlockSpec((TN, d), lambda i: (i, 0)),
        out_shape=jax.ShapeDtypeStruct((n, d), _f32),
    )(h, a.reshape(1, d), c.reshape(1, d))


def _cat_body(u0_ref, u1_ref, v0_ref, v1_ref, deg_ref, den_ref, r_ref,
              w_ref, crow_ref, o_ref):
    su = lax.rsqrt(deg_ref[...])
    sv = 1.0 / jnp.maximum(den_ref[...], 1e-16)
    dn = (((1,), (0,)), ((), ()))
    w = w_ref[...]
    acc = lax.dot_general(u0_ref[...] * su, w[0], dn, preferred_element_type=_f32)
    acc += lax.dot_general(u1_ref[...] * su, w[1], dn, preferred_element_type=_f32)
    acc += lax.dot_general(v0_ref[...] * sv, w[2], dn, preferred_element_type=_f32)
    acc += lax.dot_general(v1_ref[...] * sv, w[3], dn, preferred_element_type=_f32)
    o_ref[...] = jnp.maximum(acc + r_ref[...] + crow_ref[...], 0.0)


def _cat_layer(u0, u1, v0, v1, deg, den, r, w4, crow):
    n = r.shape[0]
    bs = lambda s: pl.BlockSpec(s, lambda i: (i, 0))
    return pl.pallas_call(
        _cat_body,
        grid=(n // TN,),
        in_specs=[
            bs((TN, 128)), bs((TN, 128)), bs((TN, 128)), bs((TN, 128)),
            bs((TN, 1)), bs((TN, 1)), bs((TN, H)),
            pl.BlockSpec((4, 128, H), lambda i: (0, 0, 0)),
            pl.BlockSpec((1, H), lambda i: (0, 0)),
        ],
        out_specs=bs((TN, H)),
        out_shape=jax.ShapeDtypeStruct((n, H), _f32),
    )(u0, u1, v0, v1, deg, den, r, w4, crow.reshape(1, H))


def _segmean_body(nblk, x_ref, bf_ref, o_ref, cnt_ref):
    i = pl.program_id(0)

    @pl.when(i == 0)
    def _():
        o_ref[...] = jnp.zeros_like(o_ref)
        cnt_ref[...] = jnp.zeros_like(cnt_ref)

    cols = lax.broadcasted_iota(_i32, (1, B), 1).astype(_f32)
    onehot = (bf_ref[...] == cols).astype(_f32)
    dn = (((0,), (0,)), ((), ()))
    o_ref[...] += lax.dot_general(onehot, x_ref[...], dn,
                                  preferred_element_type=_f32,
                                  precision=lax.Precision.HIGHEST)
    cnt_ref[...] += lax.dot_general(onehot, jnp.ones_like(x_ref[...]), dn,
                                    preferred_element_type=_f32,
                                    precision=lax.Precision.HIGHEST)

    @pl.when(i == nblk - 1)
    def _():
        o_ref[...] = o_ref[...] / jnp.maximum(cnt_ref[...], 1.0)


def _segmean(x, bf):
    n, d = x.shape
    nblk = n // TN
    return pl.pallas_call(
        functools.partial(_segmean_body, nblk),
        grid=(nblk,),
        in_specs=[
            pl.BlockSpec((TN, d), lambda i: (i, 0)),
            pl.BlockSpec((TN, 1), lambda i: (i, 0)),
        ],
        out_specs=pl.BlockSpec((B, d), lambda i: (0, 0)),
        out_shape=jax.ShapeDtypeStruct((B, d), _f32),
        scratch_shapes=[pltpu.VMEM((B, d), _f32)],
    )(x, bf)


def _colsum_body(p_ref, o_ref):
    o_ref[...] = jnp.sum(p_ref[...], axis=0, keepdims=True)[None]


def _colsum(parts):
    nt = NP // 128
    out = pl.pallas_call(
        _colsum_body,
        grid=(nt,),
        in_specs=[pl.BlockSpec((32, 128), lambda i: (0, i))],
        out_specs=pl.BlockSpec((1, 1, 128), lambda i: (i, 0, 0)),
        out_shape=jax.ShapeDtypeStruct((nt, 1, 128), _f32),
    )(parts)
    return out.reshape(NP)


def _mlp_body(xx_ref, w1_ref, w2_ref, w3_ref, o_ref):
    dn = (((1,), (0,)), ((), ()))
    h = jnp.maximum(lax.dot_general(xx_ref[...], w1_ref[...], dn,
                                    preferred_element_type=_f32), 0.0)
    h = jnp.maximum(lax.dot_general(h, w2_ref[...], dn,
                                    preferred_element_type=_f32), 0.0)
    o_ref[...] = lax.dot_general(h, w3_ref[...], dn, preferred_element_type=_f32)


def _final_mlp(xx, w1, w2, w3p):
    d = xx.shape[1]
    return pl.pallas_call(
        _mlp_body,
        in_specs=[
            pl.BlockSpec((B, d), lambda: (0, 0)),
            pl.BlockSpec((d, H), lambda: (0, 0)),
            pl.BlockSpec((H, H), lambda: (0, 0)),
            pl.BlockSpec((H, 128), lambda: (0, 0)),
        ],
        out_specs=pl.BlockSpec((B, 128), lambda: (0, 0)),
        out_shape=jax.ShapeDtypeStruct((B, 128), _f32),
    )(xx, w1, w2, w3p)


def _sc_mesh():
    return plsc.VectorSubcoreMesh(core_axis_name="c", subcore_axis_name="s",
                                  num_cores=NC, num_subcores=NS)


def _tid():
    return lax.axis_index("s") * NC + lax.axis_index("c")


def _zero_1d(ref, nwords):
    def bd(k, _):
        ref[pl.ds(k * LANES, LANES)] = jnp.zeros((LANES,), _f32)
        return 0
    lax.fori_loop(0, nwords // LANES, bd, 0)


def _deg_count(dst_t):
    nbt = dst_t.shape[1]

    def body(dst_hbm, parts_hbm, dstb, den, sem):
        tid = _tid()
        _zero_1d(den, NP)
        pltpu.sync_copy(dst_hbm.at[tid], dstb)
        ones = jnp.ones((LANES,), _f32)

        def row(r, _):
            for j in range(8):
                di = dstb[r, j * LANES:(j + 1) * LANES]
                plsc.addupdate_scatter(den, [di], ones)
            return 0
        lax.fori_loop(0, nbt, row, 0)
        pltpu.sync_copy(den, parts_hbm.at[pl.ds(tid * NP, NP)])

    return pl.kernel(
        body,
        out_type=jax.ShapeDtypeStruct((NC * NS * NP,), _f32),
        mesh=_sc_mesh(),
        compiler_params=pltpu.CompilerParams(needs_layout_passes=False),
        scratch_types=[
            pltpu.VMEM((nbt, 128), _i32),
            pltpu.VMEM((NP,), _f32),
            pltpu.SemaphoreType.DMA,
        ],
    )(dst_t)


def _gat_alpha(s_tab, d_tab, src_t, dst_t):
    nbt = src_t.shape[1]

    def body(s_hbm, d_hbm, src_hbm, dst_hbm, e_hbm, parts_hbm,
             st, dt, den, srcb, dstb, eb, sem):
        tid = _tid()
        pltpu.sync_copy(s_hbm, st)
        pltpu.sync_copy(d_hbm, dt)
        _zero_1d(den, NP)
        pltpu.sync_copy(src_hbm.at[tid], srcb)
        pltpu.sync_copy(dst_hbm.at[tid], dstb)

        def row(r, _):
            for j in range(8):
                sl = slice(j * LANES, (j + 1) * LANES)
                si = srcb[r, sl]
                di = dstb[r, sl]
                sg = plsc.load_gather(st, [si])
                dg = plsc.load_gather(dt, [di])
                z = sg + dg
                e = jnp.exp(jnp.maximum(z, 0.2 * z))
                eb[r, sl] = e
                plsc.addupdate_scatter(den, [di], e)
            return 0
        lax.fori_loop(0, nbt, row, 0)
        pltpu.sync_copy(eb, e_hbm.at[tid])
        pltpu.sync_copy(den, parts_hbm.at[pl.ds(tid * NP, NP)])

    return pl.kernel(
        body,
        out_type=[
            jax.ShapeDtypeStruct(src_t.shape, _f32),
            jax.ShapeDtypeStruct((NC * NS * NP,), _f32),
        ],
        mesh=_sc_mesh(),
        compiler_params=pltpu.CompilerParams(needs_layout_passes=False),
        scratch_types=[
            pltpu.VMEM((NP,), _f32),
            pltpu.VMEM((NP,), _f32),
            pltpu.VMEM((NP,), _f32),
            pltpu.VMEM((nbt, 128), _i32),
            pltpu.VMEM((nbt, 128), _i32),
            pltpu.VMEM((nbt, 128), _f32),
            pltpu.SemaphoreType.DMA,
        ],
    )(s_tab, d_tab, src_t, dst_t)


def _msg_scatter(tab2, src_t, dst_t, scale_t):
    nbt = src_t.shape[1]
    with_scale = scale_t is not None

    def body(*refs):
        if with_scale:
            (tab_hbm, src_hbm, dst_hbm, sc_hbm, out_hbm,
             srcb, dstb, scb, rows, acc, sem) = refs
        else:
            (tab_hbm, src_hbm, dst_hbm, out_hbm,
             srcb, dstb, rows, acc, sem) = refs
        c = lax.axis_index("c")
        s = lax.axis_index("s")

        def zrow(r, _):
            for j in range(8):
                rows[r, j * LANES:(j + 1) * LANES] = jnp.zeros((LANES,), _f32)
            return 0
        lax.fori_loop(0, 128, zrow, 0)
        base = s * RPT
        for q in range(4):
            pltpu.sync_copy(rows, acc.at[pl.ds(base + q * 128, 128)])
        pltpu.sync_copy(rows.at[pl.ds(0, RPT - 512)],
                        acc.at[pl.ds(base + 512, RPT - 512)])
        plsc.subcore_barrier()

        pltpu.sync_copy(src_hbm.at[s], srcb)
        pltpu.sync_copy(dst_hbm.at[s], dstb)
        if with_scale:
            pltpu.sync_copy(sc_hbm.at[s], scb)

        off = c * N

        def arow(r, _):
            for j in range(8):
                sl = slice(j * LANES, (j + 1) * LANES)
                srcb[r, sl] = srcb[r, sl] + off
            return 0
        lax.fori_loop(0, nbt, arow, 0)

        def blk(b, _):
            pltpu.async_copy(tab_hbm.at[srcb.at[b]], rows, sem).wait()
            if with_scale:
                def srow(k, _2):
                    cv = plsc.load_gather(
                        scb, [jnp.full((LANES,), b, _i32),
                              jnp.full((LANES,), k, _i32)])
                    for j in range(8):
                        sl = slice(j * LANES, (j + 1) * LANES)
                        rows[k, sl] = rows[k, sl] * cv
                    return 0
                lax.fori_loop(0, 128, srow, 0)
            pltpu.sync_copy(rows, acc.at[dstb.at[b]], add=True)
            return 0
        lax.fori_loop(0, nbt, blk, 0)
        plsc.subcore_barrier()

        obase = c * NROW + base
        for q in range(4):
            pltpu.sync_copy(acc.at[pl.ds(base + q * 128, 128)], rows)
            pltpu.sync_copy(rows, out_hbm.at[pl.ds(obase + q * 128, 128)])
        tail = RPT - 512
        pltpu.sync_copy(acc.at[pl.ds(base + 512, tail)], rows.at[pl.ds(0, tail)])
        pltpu.sync_copy(rows.at[pl.ds(0, tail)],
                        out_hbm.at[pl.ds(obase + 512, tail)])

    scratch = [
        pltpu.VMEM((nbt, 128), _i32),
        pltpu.VMEM((nbt, 128), _i32),
    ]
    if with_scale:
        scratch.append(pltpu.VMEM((nbt, 128), _f32))
    scratch += [
        pltpu.VMEM((128, 128), _f32),
        pltpu.VMEM_SHARED((NROW, 128), _f32),
        pltpu.SemaphoreType.DMA,
    ]
    args = (tab2, src_t, dst_t) + ((scale_t,) if with_scale else ())
    out = pl.kernel(
        body,
        out_type=jax.ShapeDtypeStruct((NC * NROW, 128), _f32),
        mesh=_sc_mesh(),
        compiler_params=pltpu.CompilerParams(needs_layout_passes=False),
        scratch_types=scratch,
    )(*args)
    return out.reshape(NC, NROW, 128)


def _pad_edges(src, dst):
    e = src.shape[0]
    ep = -(-e // EPAD) * EPAD
    src_p = jnp.concatenate([src, jnp.zeros((ep - e,), _i32)])
    dst_p = jnp.concatenate([dst, jnp.full((ep - e,), N, _i32)])
    return src_p, dst_p


def _v32(x):
    return x.reshape(NC * NS, -1, 128)


def _v16(x):
    return x.reshape(NS, -1, 128)


def _pad_tab(v):
    return jnp.concatenate([v, jnp.zeros((NP - N,), _f32)])


def _bn_affine(st, g, be):
    m = st[0] / N
    v = st[1] / N - m * m
    a = g * lax.rsqrt(v + 1e-5)
    return a, be - m * a


def kernel(x_graph_1, x_graph_2, params, edge_index_g1g1, edge_index_g2g2,
           edge_index_g1g2, edge_index_g2g1, batch_graph_1, batch_graph_2):
    nt = ["graph_1", "graph_2"]
    xin = {0: x_graph_1, 1: x_graph_2}
    bf = {0: batch_graph_1.astype(_f32).reshape(N, 1),
          1: batch_graph_2.astype(_f32).reshape(N, 1)}

    start = [_segmean(xin[t], bf[t]) for t in (0, 1)]

    x = {}
    for t in (0, 1):
        p = params["pre"][-1][nt[t]]
        h, st = _mm_stats(xin[t], p["W"], p["b"])
        a, c = _bn_affine(st, p["g"], p["be"])
        x[t] = _affine(h, a, c, relu=True)

    loop = jnp.arange(N, dtype=_i32)
    homo_edges = {}
    deg = {}
    for t, ei in ((0, edge_index_g1g1), (1, edge_index_g2g2)):
        src_p, dst_p = _pad_edges(jnp.concatenate([ei[0], loop]),
                                  jnp.concatenate([ei[1], loop]))
        homo_edges[t] = (_v16(src_p), _v16(dst_p))
        deg[t] = _colsum(_deg_count(_v32(dst_p)).reshape(NC * NS, NP))[:N].reshape(N, 1)

    cross_edges = {
        (0, 1): _pad_edges(edge_index_g1g2[0], edge_index_g1g2[1]),
        (1, 0): _pad_edges(edge_index_g2g1[0], edge_index_g2g1[1]),
    }

    for i in range(len(params["conv"])):
        pc = params["conv"][i]
        gcn_p = {0: pc["gcn_g1"], 1: pc["gcn_g2"]}
        gat_p = {(0, 1): pc["gat_12"], (1, 0): pc["gat_21"]}

        tabs = {}
        sd = {}
        for t in (0, 1):
            go = gat_p[(t, 1 - t)]
            gi = gat_p[(1 - t, t)]
            wbig = jnp.concatenate([gcn_p[t]["W"], go["Ws"], gi["Wd"]], axis=1)
            gcn_tab, gat_tab, sd_t = _conv_mm(x[t], wbig, deg[t],
                                              go["as_"], gi["ad"])
            tabs[t] = (gcn_tab.reshape(2 * N, 128), gat_tab.reshape(2 * N, 128))
            sd[t] = sd_t

        msg_gcn = {}
        msg_gat = {}
        den = {}
        for t in (0, 1):
            src_t, dst_t = homo_edges[t]
            msg_gcn[t] = _msg_scatter(tabs[t][0], src_t, dst_t, None)
        for (ts, td) in ((0, 1), (1, 0)):
            src_p, dst_p = cross_edges[(ts, td)]
            s_tab = _pad_tab(sd[ts][:, 0])
            d_tab = _pad_tab(sd[td][:, 1])
            e_t, parts = _gat_alpha(s_tab, d_tab, _v32(src_p), _v32(dst_p))
            den[td] = _colsum(parts.reshape(NC * NS, NP))[:N].reshape(N, 1)
            msg_gat[td] = _msg_scatter(tabs[ts][1], _v16(src_p), _v16(dst_p),
                                       _v16(e_t.reshape(-1)))

        for t in (0, 1):
            pcat = params["cat"][i][nt[t]]
            w4 = pcat["W"].reshape(4, 128, H)
            gi = gat_p[(1 - t, t)]
            crow = (gcn_p[t]["b"] @ pcat["W"][:H]
                    + gi["b"] @ pcat["W"][H:] + pcat["b"])
            u = msg_gcn[t]
            v = msg_gat[t]
            x[t] = _cat_layer(u[0, :N], u[1, :N], v[0, :N], v[1, :N],
                              deg[t], den[t], x[t], w4, crow)

    rep_affine = {}
    for j in range(len(params["post"])):
        last = j == len(params["post"]) - 1
        for t in (0, 1):
            p = params["post"][j][nt[t]]
            h, st = _mm_stats(x[t], p["W"], p["b"])
            a, c = _bn_affine(st, p["g"], p["be"])
            if last:
                x[t] = h
                rep_affine[t] = (a, c)
            else:
                x[t] = _affine(h, a, c, relu=True)

    reps = [_segmean(x[t], bf[t]) for t in (0, 1)]
    a1, c1 = rep_affine[0]
    a2, c2 = rep_affine[1]
    w3p = jnp.zeros((H, 128), _f32).at[:, :2].set(params["lin3"])

    xx = jnp.concatenate([start[0], start[1],
                          reps[0] * a1 + c1, reps[1] * a2 + c2], axis=1)
    out = _final_mlp(xx, params["lin1"], params["lin2"], w3p)
    return out[:, :2]


# speedup vs baseline: 9.0658x; 1.3101x over previous
import functools

import jax
import jax.numpy as jnp
from jax import lax
from jax.experimental import pallas as pl
from jax.experimental.pallas import tpu as pltpu
from jax.experimental.pallas import tpu_sc as plsc

N = 10000
B = 16
H = 256
DIN = 128
NP = 10112
NC, NS, LANES = 2, 16, 16
TN = 1000
EPAD = 4096
NROW = NP
RPT = NROW // NS

_f32 = jnp.float32
_i32 = jnp.int32


def _mm_stats_body(x_ref, w_ref, b_ref, o_ref, st_ref):
    i = pl.program_id(0)
    h = lax.dot_general(x_ref[...], w_ref[...], (((1,), (0,)), ((), ())),
                        preferred_element_type=_f32) + b_ref[...]
    o_ref[...] = h

    @pl.when(i == 0)
    def _():
        st_ref[...] = jnp.zeros_like(st_ref)

    s1 = jnp.sum(h, axis=0, keepdims=True)
    s2 = jnp.sum(h * h, axis=0, keepdims=True)
    st_ref[0:1, :] += s1
    st_ref[1:2, :] += s2


def _mm_stats(x, w, b):
    n, k = x.shape
    c = w.shape[1]
    grid = (n // TN,)
    return pl.pallas_call(
        _mm_stats_body,
        grid=grid,
        in_specs=[
            pl.BlockSpec((TN, k), lambda i: (i, 0)),
            pl.BlockSpec((k, c), lambda i: (0, 0)),
            pl.BlockSpec((1, c), lambda i: (0, 0)),
        ],
        out_specs=[
            pl.BlockSpec((TN, c), lambda i: (i, 0)),
            pl.BlockSpec((8, c), lambda i: (0, 0)),
        ],
        out_shape=[
            jax.ShapeDtypeStruct((n, c), _f32),
            jax.ShapeDtypeStruct((8, c), _f32),
        ],
    )(x, w, b.reshape(1, c))


def _conv_mm_body(x_ref, w_ref, deg_ref, asv_ref, ad_ref,
                  gcn_ref, gat_ref, sd_ref):
    h = lax.dot_general(x_ref[...], w_ref[...], (((1,), (0,)), ((), ())),
                        preferred_element_type=_f32)
    dinv = lax.rsqrt(deg_ref[...])
    hg = h[:, 0:256] * dinv
    gcn_ref[0] = hg[:, 0:128]
    gcn_ref[1] = hg[:, 128:256]
    hs = h[:, 256:512]
    gat_ref[0] = hs[:, 0:128]
    gat_ref[1] = hs[:, 128:256]
    dn = (((1,), (0,)), ((), ()))
    s = lax.dot_general(hs, asv_ref[...], dn, preferred_element_type=_f32)
    dcol = lax.dot_general(h[:, 512:768], ad_ref[...], dn,
                           preferred_element_type=_f32)
    zero = jnp.zeros((s.shape[0], 126), _f32)
    sd_ref[...] = jnp.concatenate([s, dcol, zero], axis=1)


def _conv_mm(x, wbig, deg, asv, ad):
    n = x.shape[0]
    grid = (n // TN,)
    return pl.pallas_call(
        _conv_mm_body,
        grid=grid,
        in_specs=[
            pl.BlockSpec((TN, H), lambda i: (i, 0)),
            pl.BlockSpec((H, 768), lambda i: (0, 0)),
            pl.BlockSpec((TN, 1), lambda i: (i, 0)),
            pl.BlockSpec((H, 1), lambda i: (0, 0)),
            pl.BlockSpec((H, 1), lambda i: (0, 0)),
        ],
        out_specs=[
            pl.BlockSpec((2, TN, 128), lambda i: (0, i, 0)),
            pl.BlockSpec((2, TN, 128), lambda i: (0, i, 0)),
            pl.BlockSpec((TN, 128), lambda i: (i, 0)),
        ],
        out_shape=[
            jax.ShapeDtypeStruct((2, n, 128), _f32),
            jax.ShapeDtypeStruct((2, n, 128), _f32),
            jax.ShapeDtypeStruct((n, 128), _f32),
        ],
    )(x, wbig, deg, asv.reshape(H, 1), ad.reshape(H, 1))


def _affine_body(relu, h_ref, a_ref, c_ref, o_ref):
    y = h_ref[...] * a_ref[...] + c_ref[...]
    if relu:
        y = jnp.maximum(y, 0.0)
    o_ref[...] = y


def _affine(h, a, c, relu):
    n, d = h.shape
    return pl.pallas_call(
        functools.partial(_affine_body, relu),
        grid=(n // TN,),
        in_specs=[
            pl.BlockSpec((TN, d), lambda i: (i, 0)),
            pl.BlockSpec((1, d), lambda i: (0, 0)),
            pl.BlockSpec((1, d), lambda i: (0, 0)),
        ],
        out_specs=pl.BlockSpec((TN, d), lambda i: (i, 0)),
        out_shape=jax.ShapeDtypeStruct((n, d), _f32),
    )(h, a.reshape(1, d), c.reshape(1, d))


def _cat_body(u0_ref, u1_ref, v0_ref, v1_ref, deg_ref, den_ref, r_ref,
              w_ref, crow_ref, o_ref):
    su = lax.rsqrt(deg_ref[...])
    sv = 1.0 / jnp.maximum(den_ref[...], 1e-16)
    dn = (((1,), (0,)), ((), ()))
    w = w_ref[...]
    acc = lax.dot_general(u0_ref[...] * su, w[0], dn, preferred_element_type=_f32)
    acc += lax.dot_general(u1_ref[...] * su, w[1], dn, preferred_element_type=_f32)
    acc += lax.dot_general(v0_ref[...] * sv, w[2], dn, preferred_element_type=_f32)
    acc += lax.dot_general(v1_ref[...] * sv, w[3], dn, preferred_element_type=_f32)
    o_ref[...] = jnp.maximum(acc + r_ref[...] + crow_ref[...], 0.0)


def _cat_layer(u0, u1, v0, v1, deg, den, r, w4, crow):
    n = r.shape[0]
    bs = lambda s: pl.BlockSpec(s, lambda i: (i, 0))
    return pl.pallas_call(
        _cat_body,
        grid=(n // TN,),
        in_specs=[
            bs((TN, 128)), bs((TN, 128)), bs((TN, 128)), bs((TN, 128)),
            bs((TN, 1)), bs((TN, 1)), bs((TN, H)),
            pl.BlockSpec((4, 128, H), lambda i: (0, 0, 0)),
            pl.BlockSpec((1, H), lambda i: (0, 0)),
        ],
        out_specs=bs((TN, H)),
        out_shape=jax.ShapeDtypeStruct((n, H), _f32),
    )(u0, u1, v0, v1, deg, den, r, w4, crow.reshape(1, H))


def _segmean_body(nblk, x_ref, bf_ref, o_ref, cnt_ref):
    i = pl.program_id(0)

    @pl.when(i == 0)
    def _():
        o_ref[...] = jnp.zeros_like(o_ref)
        cnt_ref[...] = jnp.zeros_like(cnt_ref)

    cols = lax.broadcasted_iota(_i32, (1, B), 1).astype(_f32)
    onehot = (bf_ref[...] == cols).astype(_f32)
    dn = (((0,), (0,)), ((), ()))
    o_ref[...] += lax.dot_general(onehot, x_ref[...], dn,
                                  preferred_element_type=_f32,
                                  precision=lax.Precision.HIGHEST)
    cnt_ref[...] += lax.dot_general(onehot, jnp.ones_like(x_ref[...]), dn,
                                    preferred_element_type=_f32,
                                    precision=lax.Precision.HIGHEST)

    @pl.when(i == nblk - 1)
    def _():
        o_ref[...] = o_ref[...] / jnp.maximum(cnt_ref[...], 1.0)


def _segmean(x, bf):
    n, d = x.shape
    nblk = n // TN
    return pl.pallas_call(
        functools.partial(_segmean_body, nblk),
        grid=(nblk,),
        in_specs=[
            pl.BlockSpec((TN, d), lambda i: (i, 0)),
            pl.BlockSpec((TN, 1), lambda i: (i, 0)),
        ],
        out_specs=pl.BlockSpec((B, d), lambda i: (0, 0)),
        out_shape=jax.ShapeDtypeStruct((B, d), _f32),
        scratch_shapes=[pltpu.VMEM((B, d), _f32)],
    )(x, bf)


def _colsum_body(p_ref, o_ref):
    o_ref[...] = jnp.sum(p_ref[...], axis=0, keepdims=True)[None]


def _colsum(parts):
    nt = NP // 128
    out = pl.pallas_call(
        _colsum_body,
        grid=(nt,),
        in_specs=[pl.BlockSpec((32, 128), lambda i: (0, i))],
        out_specs=pl.BlockSpec((1, 1, 128), lambda i: (i, 0, 0)),
        out_shape=jax.ShapeDtypeStruct((nt, 1, 128), _f32),
    )(parts)
    return out.reshape(NP)


def _mlp_body(xx_ref, w1_ref, w2_ref, w3_ref, o_ref):
    dn = (((1,), (0,)), ((), ()))
    h = jnp.maximum(lax.dot_general(xx_ref[...], w1_ref[...], dn,
                                    preferred_element_type=_f32), 0.0)
    h = jnp.maximum(lax.dot_general(h, w2_ref[...], dn,
                                    preferred_element_type=_f32), 0.0)
    o_ref[...] = lax.dot_general(h, w3_ref[...], dn, preferred_element_type=_f32)


def _final_mlp(xx, w1, w2, w3p):
    d = xx.shape[1]
    return pl.pallas_call(
        _mlp_body,
        in_specs=[
            pl.BlockSpec((B, d), lambda: (0, 0)),
            pl.BlockSpec((d, H), lambda: (0, 0)),
            pl.BlockSpec((H, H), lambda: (0, 0)),
            pl.BlockSpec((H, 128), lambda: (0, 0)),
        ],
        out_specs=pl.BlockSpec((B, 128), lambda: (0, 0)),
        out_shape=jax.ShapeDtypeStruct((B, 128), _f32),
    )(xx, w1, w2, w3p)


def _sc_mesh():
    return plsc.VectorSubcoreMesh(core_axis_name="c", subcore_axis_name="s",
                                  num_cores=NC, num_subcores=NS)


def _tid():
    return lax.axis_index("s") * NC + lax.axis_index("c")


def _zero_1d(ref, nwords):
    def bd(k, _):
        ref[pl.ds(k * LANES, LANES)] = jnp.zeros((LANES,), _f32)
        return 0
    lax.fori_loop(0, nwords // LANES, bd, 0)


def _deg_count(dst_t):
    nbt = dst_t.shape[1]

    def body(dst_hbm, parts_hbm, dstb, den, sem):
        tid = _tid()
        _zero_1d(den, NP)
        pltpu.sync_copy(dst_hbm.at[tid], dstb)
        ones = jnp.ones((LANES,), _f32)

        def row(r, _):
            for j in range(8):
                di = dstb[r, j * LANES:(j + 1) * LANES]
                plsc.addupdate_scatter(den, [di], ones)
            return 0
        lax.fori_loop(0, nbt, row, 0)
        pltpu.sync_copy(den, parts_hbm.at[pl.ds(tid * NP, NP)])

    return pl.kernel(
        body,
        out_type=jax.ShapeDtypeStruct((NC * NS * NP,), _f32),
        mesh=_sc_mesh(),
        compiler_params=pltpu.CompilerParams(needs_layout_passes=False),
        scratch_types=[
            pltpu.VMEM((nbt, 128), _i32),
            pltpu.VMEM((NP,), _f32),
            pltpu.SemaphoreType.DMA,
        ],
    )(dst_t)


def _gat_alpha(s_tab, d_tab, src_t, dst_t):
    nbt = src_t.shape[1]

    def body(s_hbm, d_hbm, src_hbm, dst_hbm, e_hbm, parts_hbm,
             st, dt, den, srcb, dstb, eb, sem):
        tid = _tid()
        pltpu.sync_copy(s_hbm, st)
        pltpu.sync_copy(d_hbm, dt)
        _zero_1d(den, NP)
        pltpu.sync_copy(src_hbm.at[tid], srcb)
        pltpu.sync_copy(dst_hbm.at[tid], dstb)

        def row(r, _):
            for j in range(8):
                sl = slice(j * LANES, (j + 1) * LANES)
                si = srcb[r, sl]
                di = dstb[r, sl]
                sg = plsc.load_gather(st, [si])
                dg = plsc.load_gather(dt, [di])
                z = sg + dg
                e = jnp.exp(jnp.maximum(z, 0.2 * z))
                eb[r, sl] = e
                plsc.addupdate_scatter(den, [di], e)
            return 0
        lax.fori_loop(0, nbt, row, 0)
        pltpu.sync_copy(eb, e_hbm.at[tid])
        pltpu.sync_copy(den, parts_hbm.at[pl.ds(tid * NP, NP)])

    return pl.kernel(
        body,
        out_type=[
            jax.ShapeDtypeStruct(src_t.shape, _f32),
            jax.ShapeDtypeStruct((NC * NS * NP,), _f32),
        ],
        mesh=_sc_mesh(),
        compiler_params=pltpu.CompilerParams(needs_layout_passes=False),
        scratch_types=[
            pltpu.VMEM((NP,), _f32),
            pltpu.VMEM((NP,), _f32),
            pltpu.VMEM((NP,), _f32),
            pltpu.VMEM((nbt, 128), _i32),
            pltpu.VMEM((nbt, 128), _i32),
            pltpu.VMEM((nbt, 128), _f32),
            pltpu.SemaphoreType.DMA,
        ],
    )(s_tab, d_tab, src_t, dst_t)


def _msg_scatter(tab2, srcoff_t, dst_flat, scale_t):
    nbt = srcoff_t.shape[1]
    with_scale = scale_t is not None

    def body(*refs):
        if with_scale:
            (tab_hbm, src_hbm, dst_hbm, sc_hbm, out_hbm,
             srcb, scb, dxa, dxb, rwa, rwb, acc, gsa, gsb, dsa, dsb) = refs
        else:
            (tab_hbm, src_hbm, dst_hbm, out_hbm,
             srcb, dxa, dxb, rwa, rwb, acc, gsa, gsb, dsa, dsb) = refs
        c = lax.axis_index("c")
        s = lax.axis_index("s")
        wid = c * NS + s

        def zrow(r, _):
            for j in range(8):
                rwa[r, j * LANES:(j + 1) * LANES] = jnp.zeros((LANES,), _f32)
            return 0
        lax.fori_loop(0, 128, zrow, 0)
        base = s * RPT
        for q in range(4):
            pltpu.sync_copy(rwa, acc.at[pl.ds(base + q * 128, 128)])
        pltpu.sync_copy(rwa.at[pl.ds(0, RPT - 512)],
                        acc.at[pl.ds(base + 512, RPT - 512)])
        plsc.subcore_barrier()

        pltpu.sync_copy(src_hbm.at[wid], srcb)
        if with_scale:
            pltpu.sync_copy(sc_hbm.at[s], scb)
        ebase = s * nbt * 128

        def start(b, didx, rows, gsem, dsem):
            pltpu.async_copy(dst_hbm.at[pl.ds(ebase + b * 128, 128)],
                             didx, dsem)
            pltpu.async_copy(tab_hbm.at[srcb.at[b]], rows, gsem)

        def finish(b, didx, rows, gsem, dsem):
            pltpu.make_async_copy(tab_hbm.at[srcb.at[b]], rows, gsem).wait()
            if with_scale:
                def srow(k, _2):
                    cv = plsc.load_gather(
                        scb, [jnp.full((LANES,), b, _i32),
                              jnp.full((LANES,), k, _i32)])
                    for j in range(8):
                        sl = slice(j * LANES, (j + 1) * LANES)
                        rows[k, sl] = rows[k, sl] * cv
                    return 0
                lax.fori_loop(0, 128, srow, 0)
            pltpu.make_async_copy(dst_hbm.at[pl.ds(ebase + b * 128, 128)],
                                  didx, dsem).wait()
            pltpu.sync_copy(rows, acc.at[didx], add=True)

        start(0, dxa, rwa, gsa, dsa)

        def pair(k, _):
            b0 = 2 * k
            start(b0 + 1, dxb, rwb, gsb, dsb)
            finish(b0, dxa, rwa, gsa, dsa)
            start(b0 + 2, dxa, rwa, gsa, dsa)
            finish(b0 + 1, dxb, rwb, gsb, dsb)
            return 0
        lax.fori_loop(0, nbt // 2 - 1, pair, 0)
        bl = nbt - 2
        start(bl + 1, dxb, rwb, gsb, dsb)
        finish(bl, dxa, rwa, gsa, dsa)
        finish(bl + 1, dxb, rwb, gsb, dsb)
        plsc.subcore_barrier()

        obase = c * NROW + base
        for q in range(4):
            pltpu.sync_copy(acc.at[pl.ds(base + q * 128, 128)], rwa)
            pltpu.sync_copy(rwa, out_hbm.at[pl.ds(obase + q * 128, 128)])
        tail = RPT - 512
        pltpu.sync_copy(acc.at[pl.ds(base + 512, tail)], rwa.at[pl.ds(0, tail)])
        pltpu.sync_copy(rwa.at[pl.ds(0, tail)],
                        out_hbm.at[pl.ds(obase + 512, tail)])

    scratch = [pltpu.VMEM((nbt, 128), _i32)]
    if with_scale:
        scratch.append(pltpu.VMEM((nbt, 128), _f32))
    scratch += [
        pltpu.VMEM((128,), _i32),
        pltpu.VMEM((128,), _i32),
        pltpu.VMEM((128, 128), _f32),
        pltpu.VMEM((128, 128), _f32),
        pltpu.VMEM_SHARED((NROW, 128), _f32),
        pltpu.SemaphoreType.DMA,
        pltpu.SemaphoreType.DMA,
        pltpu.SemaphoreType.DMA,
        pltpu.SemaphoreType.DMA,
    ]
    args = (tab2, srcoff_t, dst_flat) + ((scale_t,) if with_scale else ())
    out = pl.kernel(
        body,
        out_type=jax.ShapeDtypeStruct((NC * NROW, 128), _f32),
        mesh=_sc_mesh(),
        compiler_params=pltpu.CompilerParams(needs_layout_passes=False),
        scratch_types=scratch,
    )(*args)
    return out.reshape(NC, NROW, 128)


def _pad_edges(src, dst):
    e = src.shape[0]
    ep = -(-e // EPAD) * EPAD
    src_p = jnp.concatenate([src, jnp.zeros((ep - e,), _i32)])
    dst_p = jnp.concatenate([dst, jnp.full((ep - e,), N, _i32)])
    return src_p, dst_p


def _v32(x):
    return x.reshape(NC * NS, -1, 128)


def _v16(x):
    return x.reshape(NS, -1, 128)


def _vsrc(src_p):
    return jnp.stack([src_p, src_p + N]).reshape(NC * NS, -1, 128)


def _pad_tab(v):
    return jnp.concatenate([v, jnp.zeros((NP - N,), _f32)])


def _bn_affine(st, g, be):
    m = st[0] / N
    v = st[1] / N - m * m
    a = g * lax.rsqrt(v + 1e-5)
    return a, be - m * a


def kernel(x_graph_1, x_graph_2, params, edge_index_g1g1, edge_index_g2g2,
           edge_index_g1g2, edge_index_g2g1, batch_graph_1, batch_graph_2):
    nt = ["graph_1", "graph_2"]
    xin = {0: x_graph_1, 1: x_graph_2}
    bf = {0: batch_graph_1.astype(_f32).reshape(N, 1),
          1: batch_graph_2.astype(_f32).reshape(N, 1)}

    start = [_segmean(xin[t], bf[t]) for t in (0, 1)]

    x = {}
    for t in (0, 1):
        p = params["pre"][-1][nt[t]]
        h, st = _mm_stats(xin[t], p["W"], p["b"])
        a, c = _bn_affine(st, p["g"], p["be"])
        x[t] = _affine(h, a, c, relu=True)

    loop = jnp.arange(N, dtype=_i32)
    homo_edges = {}
    deg = {}
    for t, ei in ((0, edge_index_g1g1), (1, edge_index_g2g2)):
        src_p, dst_p = _pad_edges(jnp.concatenate([ei[0], loop]),
                                  jnp.concatenate([ei[1], loop]))
        homo_edges[t] = (_vsrc(src_p), dst_p)
        deg[t] = _colsum(_deg_count(_v32(dst_p)).reshape(NC * NS, NP))[:N].reshape(N, 1)

    cross_edges = {
        (0, 1): _pad_edges(edge_index_g1g2[0], edge_index_g1g2[1]),
        (1, 0): _pad_edges(edge_index_g2g1[0], edge_index_g2g1[1]),
    }

    for i in range(len(params["conv"])):
        pc = params["conv"][i]
        gcn_p = {0: pc["gcn_g1"], 1: pc["gcn_g2"]}
        gat_p = {(0, 1): pc["gat_12"], (1, 0): pc["gat_21"]}

        tabs = {}
        sd = {}
        for t in (0, 1):
            go = gat_p[(t, 1 - t)]
            gi = gat_p[(1 - t, t)]
            wbig = jnp.concatenate([gcn_p[t]["W"], go["Ws"], gi["Wd"]], axis=1)
            gcn_tab, gat_tab, sd_t = _conv_mm(x[t], wbig, deg[t],
                                              go["as_"], gi["ad"])
            tabs[t] = (gcn_tab.reshape(2 * N, 128), gat_tab.reshape(2 * N, 128))
            sd[t] = sd_t

        msg_gcn = {}
        msg_gat = {}
        den = {}
        for t in (0, 1):
            src_t, dst_t = homo_edges[t]
            msg_gcn[t] = _msg_scatter(tabs[t][0], src_t, dst_t, None)
        for (ts, td) in ((0, 1), (1, 0)):
            src_p, dst_p = cross_edges[(ts, td)]
            s_tab = _pad_tab(sd[ts][:, 0])
            d_tab = _pad_tab(sd[td][:, 1])
            e_t, parts = _gat_alpha(s_tab, d_tab, _v32(src_p), _v32(dst_p))
            den[td] = _colsum(parts.reshape(NC * NS, NP))[:N].reshape(N, 1)
            msg_gat[td] = _msg_scatter(tabs[ts][1], _vsrc(src_p), dst_p,
                                       _v16(e_t.reshape(-1)))

        for t in (0, 1):
            pcat = params["cat"][i][nt[t]]
            w4 = pcat["W"].reshape(4, 128, H)
            gi = gat_p[(1 - t, t)]
            crow = (gcn_p[t]["b"] @ pcat["W"][:H]
                    + gi["b"] @ pcat["W"][H:] + pcat["b"])
            u = msg_gcn[t]
            v = msg_gat[t]
            x[t] = _cat_layer(u[0, :N], u[1, :N], v[0, :N], v[1, :N],
                              deg[t], den[t], x[t], w4, crow)

    rep_affine = {}
    for j in range(len(params["post"])):
        last = j == len(params["post"]) - 1
        for t in (0, 1):
            p = params["post"][j][nt[t]]
            h, st = _mm_stats(x[t], p["W"], p["b"])
            a, c = _bn_affine(st, p["g"], p["be"])
            if last:
                x[t] = h
                rep_affine[t] = (a, c)
            else:
                x[t] = _affine(h, a, c, relu=True)

    reps = [_segmean(x[t], bf[t]) for t in (0, 1)]
    a1, c1 = rep_affine[0]
    a2, c2 = rep_affine[1]
    w3p = jnp.zeros((H, 128), _f32).at[:, :2].set(params["lin3"])

    xx = jnp.concatenate([start[0], start[1],
                          reps[0] * a1 + c1, reps[1] * a2 + c2], axis=1)
    out = _final_mlp(xx, params["lin1"], params["lin2"], w3p)
    return out[:, :2]
